# baseline (speedup 1.0000x reference)
import functools

import jax
import jax.numpy as jnp
from jax import lax
from jax.experimental import pallas as pl
from jax.experimental.pallas import tpu as pltpu
from jax.experimental.pallas import tpu_sc as plsc

N = 10000
NP = 10240
E = 320000
D = 128
DA = 144

NC = 2
NS = 16
NW = NC * NS
K = 32
EP = 327680
CHUNKS = EP // K // NW
QCH = 32
NBUF = 4
CH0 = 160
CH1 = 2 * CHUNKS - CH0
ROWS_PER_SUB = NP // NS


def _sc_agg_body(x_hbm, src_hbm, dst_hbm, z_hbm,
                 agg_out,
                 agg_sp, sidx, didx, rows, sem_g, sem_s):
    c = lax.axis_index("c")
    s = lax.axis_index("s")
    wid = c * NS + s
    r0 = s * ROWS_PER_SUB
    npiece = ROWS_PER_SUB // K

    pltpu.sync_copy(z_hbm, rows[0])
    for p in range(npiece):
        pltpu.sync_copy(rows[0], agg_sp.at[pl.ds(r0 + p * K, K)])
    plsc.subcore_barrier()

    def run_ring(first, nslab):
        gd = [None] * NBUF
        sd = [None] * NBUF
        pltpu.sync_copy(src_hbm.at[pl.ds(first, QCH)], sidx[0])
        pltpu.sync_copy(dst_hbm.at[pl.ds(first, QCH)], didx[0])
        for b in range(NBUF):
            gd[b] = pltpu.async_copy(x_hbm.at[sidx[0].at[b]], rows[b], sem_g[b])
        for h in range(nslab):
            p, pn = h % 2, (h + 1) % 2
            if h < nslab - 1:
                base = first + (h + 1) * QCH
                pltpu.sync_copy(src_hbm.at[pl.ds(base, QCH)], sidx[pn])
                pltpu.sync_copy(dst_hbm.at[pl.ds(base, QCH)], didx[pn])
            for i in range(QCH // NBUF):
                for b in range(NBUF):
                    gd[b].wait()
                    sd[b] = pltpu.async_copy(
                        rows[b], agg_sp.at[didx[p].at[NBUF * i + b]],
                        sem_s[b], add=True)
                for b in range(NBUF):
                    sd[b].wait()
                    if i < QCH // NBUF - 1:
                        gd[b] = pltpu.async_copy(
                            x_hbm.at[sidx[p].at[NBUF * (i + 1) + b]], rows[b],
                            sem_g[b])
                    elif h < nslab - 1:
                        gd[b] = pltpu.async_copy(
                            x_hbm.at[sidx[pn].at[b]], rows[b], sem_g[b])

    @pl.when(c == 0)
    def _():
        run_ring(s * CH0, CH0 // QCH)

    @pl.when(c == 1)
    def _():
        run_ring(NS * CH0 + s * CH1, CH1 // QCH)

    plsc.subcore_barrier()
    for p in range(npiece):
        pltpu.sync_copy(agg_sp.at[pl.ds(r0 + p * K, K)], rows[0])
        pltpu.sync_copy(rows[0], agg_out.at[c, pl.ds(r0 + p * K, K)])


_sc_agg = functools.partial(
    pl.kernel,
    out_type=jax.ShapeDtypeStruct((NC, NP, DA), jnp.float32),
    mesh=plsc.VectorSubcoreMesh(core_axis_name="c", subcore_axis_name="s"),
    scratch_types=[
        pltpu.VMEM_SHARED((NP, DA), jnp.float32),
        [pltpu.VMEM((QCH, K), jnp.int32)] * 2,
        [pltpu.VMEM((QCH, K), jnp.int32)] * 2,
        [pltpu.VMEM((K, DA), jnp.float32)] * NBUF,
        [pltpu.SemaphoreType.DMA] * NBUF,
        [pltpu.SemaphoreType.DMA] * NBUF,
    ],
    compiler_params=pltpu.CompilerParams(use_tc_tiling_on_sc=False),
)(_sc_agg_body)


def _dense_body(aggp_ref, x_ref, wl_ref, wr_ref, b_ref, out_ref, *, activation):
    aggp = aggp_ref[0] + aggp_ref[1]
    agg = aggp[:, :D]
    cnt = aggp[:, D:D + 1]
    mean = agg / jnp.maximum(cnt, 1.0)
    y = (jnp.dot(mean, wl_ref[...], preferred_element_type=jnp.float32)
         + jnp.dot(x_ref[:, :D], wr_ref[...], preferred_element_type=jnp.float32)
         + b_ref[...])
    if activation == "relu":
        h = jnp.maximum(y, 0.0)
        col = lax.broadcasted_iota(jnp.int32, (y.shape[0], DA - D), 1)
        out_ref[...] = jnp.concatenate(
            [h, jnp.where(col == 0, 1.0, 0.0)], axis=1)
    else:
        m = jnp.max(y, axis=1, keepdims=True)
        e = jnp.exp(y - m)
        out_ref[...] = e / jnp.sum(e, axis=1, keepdims=True)


def _dense(aggp, x, wl_t, wr_t, b, activation):
    br = 400
    grid = N // br
    d_out = DA if activation == "relu" else D
    return pl.pallas_call(
        functools.partial(_dense_body, activation=activation),
        grid=(grid,),
        in_specs=[
            pl.BlockSpec((NC, br, DA), lambda i: (0, i, 0)),
            pl.BlockSpec((br, DA), lambda i: (i, 0)),
            pl.BlockSpec((D, D), lambda i: (0, 0)),
            pl.BlockSpec((D, D), lambda i: (0, 0)),
            pl.BlockSpec((1, D), lambda i: (0, 0)),
        ],
        out_specs=pl.BlockSpec((br, d_out), lambda i: (i, 0)),
        out_shape=jax.ShapeDtypeStruct((N, d_out), jnp.float32),
    )(aggp, x, wl_t, wr_t, b)


def _pad_edges(edge_index):
    pad = EP - E
    src = jnp.concatenate([edge_index[0], jnp.zeros((pad,), edge_index.dtype)])
    dst = jnp.concatenate([edge_index[1], jnp.full((pad,), N, edge_index.dtype)])
    return src.reshape(NW * CHUNKS, K), dst.reshape(NW * CHUNKS, K)


def kernel(x, edge_index1, edge_index2, W_l1, b_l1, W_r1, W_l2, b_l2, W_r2):
    src1, dst1 = _pad_edges(edge_index1)
    src2, dst2 = _pad_edges(edge_index2)
    zpad = jnp.zeros((K, DA), jnp.float32)
    x_aug = jnp.concatenate(
        [x, jnp.ones((N, 1), jnp.float32), jnp.zeros((N, DA - D - 1), jnp.float32)],
        axis=1)

    agg1 = _sc_agg(x_aug, src1, dst1, zpad)
    h_aug = _dense(agg1, x_aug, W_l1.T, W_r1.T, b_l1.reshape(1, D), "relu")
    agg2 = _sc_agg(h_aug, src2, dst2, zpad)
    return _dense(agg2, h_aug, W_l2.T, W_r2.T, b_l2.reshape(1, D), "softmax")


# speedup vs baseline: 1.1779x; 1.1779x over previous
import functools

import jax
import jax.numpy as jnp
from jax import lax
from jax.experimental import pallas as pl
from jax.experimental.pallas import tpu as pltpu
from jax.experimental.pallas import tpu_sc as plsc

N = 10000
NP = 10240
E = 320000
D = 128
DA = 144

NC = 2
NS = 16
NW = NC * NS
K = 32
EP = 327680
CHUNKS = EP // K // NW
QCH = 32
NBUF = 4
CH0 = 480
CH1 = 2 * CHUNKS - CH0
ROWS_PER_SUB = NP // NS


def _sc_agg_body(x_hbm, src_hbm, dst_hbm, z_hbm,
                 agg_out,
                 agg_sp, sidx, didx, rows, sem_g, sem_s):
    c = lax.axis_index("c")
    s = lax.axis_index("s")
    wid = c * NS + s
    r0 = s * ROWS_PER_SUB
    npiece = ROWS_PER_SUB // K

    pltpu.sync_copy(z_hbm, rows[0])
    for p in range(npiece):
        pltpu.sync_copy(rows[0], agg_sp.at[pl.ds(r0 + p * K, K)])
    plsc.subcore_barrier()

    def run_ring(first, nslab):
        gd = [None] * NBUF
        sd = [None] * NBUF
        pltpu.sync_copy(src_hbm.at[pl.ds(first, QCH)], sidx[0])
        pltpu.sync_copy(dst_hbm.at[pl.ds(first, QCH)], didx[0])
        for b in range(NBUF):
            gd[b] = pltpu.async_copy(x_hbm.at[sidx[0].at[b]], rows[b], sem_g[b])
        for h in range(nslab):
            p, pn = h % 2, (h + 1) % 2
            if h < nslab - 1:
                base = first + (h + 1) * QCH
                pltpu.sync_copy(src_hbm.at[pl.ds(base, QCH)], sidx[pn])
                pltpu.sync_copy(dst_hbm.at[pl.ds(base, QCH)], didx[pn])
            for i in range(QCH // NBUF):
                for b in range(NBUF):
                    gd[b].wait()
                    sd[b] = pltpu.async_copy(
                        rows[b], agg_sp.at[didx[p].at[NBUF * i + b]],
                        sem_s[b], add=True)
                for b in range(NBUF):
                    sd[b].wait()
                    if i < QCH // NBUF - 1:
                        gd[b] = pltpu.async_copy(
                            x_hbm.at[sidx[p].at[NBUF * (i + 1) + b]], rows[b],
                            sem_g[b])
                    elif h < nslab - 1:
                        gd[b] = pltpu.async_copy(
                            x_hbm.at[sidx[pn].at[b]], rows[b], sem_g[b])

    @pl.when(c == 0)
    def _():
        run_ring(s * CH0, CH0 // QCH)

    @pl.when(c == 1)
    def _():
        run_ring(NS * CH0 + s * CH1, CH1 // QCH)

    plsc.subcore_barrier()
    for p in range(npiece):
        pltpu.sync_copy(agg_sp.at[pl.ds(r0 + p * K, K)], rows[0])
        pltpu.sync_copy(rows[0], agg_out.at[c, pl.ds(r0 + p * K, K)])


_sc_agg = functools.partial(
    pl.kernel,
    out_type=jax.ShapeDtypeStruct((NC, NP, DA), jnp.float32),
    mesh=plsc.VectorSubcoreMesh(core_axis_name="c", subcore_axis_name="s"),
    scratch_types=[
        pltpu.VMEM_SHARED((NP, DA), jnp.float32),
        [pltpu.VMEM((QCH, K), jnp.int32)] * 2,
        [pltpu.VMEM((QCH, K), jnp.int32)] * 2,
        [pltpu.VMEM((K, DA), jnp.float32)] * NBUF,
        [pltpu.SemaphoreType.DMA] * NBUF,
        [pltpu.SemaphoreType.DMA] * NBUF,
    ],
    compiler_params=pltpu.CompilerParams(use_tc_tiling_on_sc=False),
)(_sc_agg_body)


def _dense_body(aggp_ref, x_ref, wl_ref, wr_ref, b_ref, out_ref, *, activation):
    aggp = aggp_ref[0] + aggp_ref[1]
    agg = aggp[:, :D]
    cnt = aggp[:, D:D + 1]
    mean = agg / jnp.maximum(cnt, 1.0)
    y = (jnp.dot(mean, wl_ref[...], preferred_element_type=jnp.float32)
         + jnp.dot(x_ref[:, :D], wr_ref[...], preferred_element_type=jnp.float32)
         + b_ref[...])
    if activation == "relu":
        h = jnp.maximum(y, 0.0)
        col = lax.broadcasted_iota(jnp.int32, (y.shape[0], DA - D), 1)
        out_ref[...] = jnp.concatenate(
            [h, jnp.where(col == 0, 1.0, 0.0)], axis=1)
    else:
        m = jnp.max(y, axis=1, keepdims=True)
        e = jnp.exp(y - m)
        out_ref[...] = e / jnp.sum(e, axis=1, keepdims=True)


def _dense(aggp, x, wl_t, wr_t, b, activation):
    br = 400
    grid = N // br
    d_out = DA if activation == "relu" else D
    return pl.pallas_call(
        functools.partial(_dense_body, activation=activation),
        grid=(grid,),
        in_specs=[
            pl.BlockSpec((NC, br, DA), lambda i: (0, i, 0)),
            pl.BlockSpec((br, DA), lambda i: (i, 0)),
            pl.BlockSpec((D, D), lambda i: (0, 0)),
            pl.BlockSpec((D, D), lambda i: (0, 0)),
            pl.BlockSpec((1, D), lambda i: (0, 0)),
        ],
        out_specs=pl.BlockSpec((br, d_out), lambda i: (i, 0)),
        out_shape=jax.ShapeDtypeStruct((N, d_out), jnp.float32),
    )(aggp, x, wl_t, wr_t, b)


def _pad_edges(edge_index):
    pad = EP - E
    src = jnp.concatenate([edge_index[0], jnp.zeros((pad,), edge_index.dtype)])
    dst = jnp.concatenate([edge_index[1], jnp.full((pad,), N, edge_index.dtype)])
    return src.reshape(NW * CHUNKS, K), dst.reshape(NW * CHUNKS, K)


def kernel(x, edge_index1, edge_index2, W_l1, b_l1, W_r1, W_l2, b_l2, W_r2):
    src1, dst1 = _pad_edges(edge_index1)
    src2, dst2 = _pad_edges(edge_index2)
    zpad = jnp.zeros((K, DA), jnp.float32)
    x_aug = jnp.concatenate(
        [x, jnp.ones((N, 1), jnp.float32), jnp.zeros((N, DA - D - 1), jnp.float32)],
        axis=1)

    agg1 = _sc_agg(x_aug, src1, dst1, zpad)
    h_aug = _dense(agg1, x_aug, W_l1.T, W_r1.T, b_l1.reshape(1, D), "relu")
    agg2 = _sc_agg(h_aug, src2, dst2, zpad)
    return _dense(agg2, h_aug, W_l2.T, W_r2.T, b_l2.reshape(1, D), "softmax")


# speedup vs baseline: 1.2293x; 1.0436x over previous
import functools

import jax
import jax.numpy as jnp
from jax import lax
from jax.experimental import pallas as pl
from jax.experimental.pallas import tpu as pltpu
from jax.experimental.pallas import tpu_sc as plsc

N = 10000
NP = 10240
E = 320000
D = 128
DA = 144

NC = 2
NS = 16
NW = NC * NS
K = 32
EP = 327680
CHUNKS = EP // K // NW
QCH = 32
NBUF = 4
CH0 = 544
CH1 = 2 * CHUNKS - CH0
ROWS_PER_SUB = NP // NS


def _sc_agg_body(x_hbm, src_hbm, dst_hbm, z_hbm,
                 agg_out,
                 agg_sp, sidx, didx, rows, sem_g, sem_s):
    c = lax.axis_index("c")
    s = lax.axis_index("s")
    wid = c * NS + s
    r0 = s * ROWS_PER_SUB
    npiece = ROWS_PER_SUB // K

    pltpu.sync_copy(z_hbm, rows[0])
    for p in range(npiece):
        pltpu.sync_copy(rows[0], agg_sp.at[pl.ds(r0 + p * K, K)])
    plsc.subcore_barrier()

    def run_ring(first, nslab):
        gd = [None] * NBUF
        sd = [None] * NBUF
        pltpu.sync_copy(src_hbm.at[pl.ds(first, QCH)], sidx[0])
        pltpu.sync_copy(dst_hbm.at[pl.ds(first, QCH)], didx[0])
        for b in range(NBUF):
            gd[b] = pltpu.async_copy(x_hbm.at[sidx[0].at[b]], rows[b], sem_g[b])
        for h in range(nslab):
            p, pn = h % 2, (h + 1) % 2
            if h < nslab - 1:
                base = first + (h + 1) * QCH
                pltpu.sync_copy(src_hbm.at[pl.ds(base, QCH)], sidx[pn])
                pltpu.sync_copy(dst_hbm.at[pl.ds(base, QCH)], didx[pn])
            for i in range(QCH // NBUF):
                for b in range(NBUF):
                    gd[b].wait()
                    sd[b] = pltpu.async_copy(
                        rows[b], agg_sp.at[didx[p].at[NBUF * i + b]],
                        sem_s[b], add=True)
                for b in range(NBUF):
                    sd[b].wait()
                    if i < QCH // NBUF - 1:
                        gd[b] = pltpu.async_copy(
                            x_hbm.at[sidx[p].at[NBUF * (i + 1) + b]], rows[b],
                            sem_g[b])
                    elif h < nslab - 1:
                        gd[b] = pltpu.async_copy(
                            x_hbm.at[sidx[pn].at[b]], rows[b], sem_g[b])

    @pl.when(c == 0)
    def _():
        run_ring(s * CH0, CH0 // QCH)

    @pl.when(c == 1)
    def _():
        run_ring(NS * CH0 + s * CH1, CH1 // QCH)

    plsc.subcore_barrier()
    for p in range(npiece):
        pltpu.sync_copy(agg_sp.at[pl.ds(r0 + p * K, K)], rows[0])
        pltpu.sync_copy(rows[0], agg_out.at[c, pl.ds(r0 + p * K, K)])


_sc_agg = functools.partial(
    pl.kernel,
    out_type=jax.ShapeDtypeStruct((NC, NP, DA), jnp.float32),
    mesh=plsc.VectorSubcoreMesh(core_axis_name="c", subcore_axis_name="s"),
    scratch_types=[
        pltpu.VMEM_SHARED((NP, DA), jnp.float32),
        [pltpu.VMEM((QCH, K), jnp.int32)] * 2,
        [pltpu.VMEM((QCH, K), jnp.int32)] * 2,
        [pltpu.VMEM((K, DA), jnp.float32)] * NBUF,
        [pltpu.SemaphoreType.DMA] * NBUF,
        [pltpu.SemaphoreType.DMA] * NBUF,
    ],
    compiler_params=pltpu.CompilerParams(use_tc_tiling_on_sc=False),
)(_sc_agg_body)


def _dense_body(aggp_ref, x_ref, wl_ref, wr_ref, b_ref, out_ref, *, activation):
    aggp = aggp_ref[0] + aggp_ref[1]
    agg = aggp[:, :D]
    cnt = aggp[:, D:D + 1]
    mean = agg / jnp.maximum(cnt, 1.0)
    y = (jnp.dot(mean, wl_ref[...], preferred_element_type=jnp.float32)
         + jnp.dot(x_ref[:, :D], wr_ref[...], preferred_element_type=jnp.float32)
         + b_ref[...])
    if activation == "relu":
        h = jnp.maximum(y, 0.0)
        col = lax.broadcasted_iota(jnp.int32, (y.shape[0], DA - D), 1)
        out_ref[...] = jnp.concatenate(
            [h, jnp.where(col == 0, 1.0, 0.0)], axis=1)
    else:
        m = jnp.max(y, axis=1, keepdims=True)
        e = jnp.exp(y - m)
        out_ref[...] = e / jnp.sum(e, axis=1, keepdims=True)


def _dense(aggp, x, wl_t, wr_t, b, activation):
    br = 400
    grid = N // br
    d_out = DA if activation == "relu" else D
    return pl.pallas_call(
        functools.partial(_dense_body, activation=activation),
        grid=(grid,),
        in_specs=[
            pl.BlockSpec((NC, br, DA), lambda i: (0, i, 0)),
            pl.BlockSpec((br, DA), lambda i: (i, 0)),
            pl.BlockSpec((D, D), lambda i: (0, 0)),
            pl.BlockSpec((D, D), lambda i: (0, 0)),
            pl.BlockSpec((1, D), lambda i: (0, 0)),
        ],
        out_specs=pl.BlockSpec((br, d_out), lambda i: (i, 0)),
        out_shape=jax.ShapeDtypeStruct((N, d_out), jnp.float32),
    )(aggp, x, wl_t, wr_t, b)


def _pad_edges(edge_index):
    pad = EP - E
    src = jnp.concatenate([edge_index[0], jnp.zeros((pad,), edge_index.dtype)])
    dst = jnp.concatenate([edge_index[1], jnp.full((pad,), N, edge_index.dtype)])
    return src.reshape(NW * CHUNKS, K), dst.reshape(NW * CHUNKS, K)


def kernel(x, edge_index1, edge_index2, W_l1, b_l1, W_r1, W_l2, b_l2, W_r2):
    src1, dst1 = _pad_edges(edge_index1)
    src2, dst2 = _pad_edges(edge_index2)
    zpad = jnp.zeros((K, DA), jnp.float32)
    x_aug = jnp.concatenate(
        [x, jnp.ones((N, 1), jnp.float32), jnp.zeros((N, DA - D - 1), jnp.float32)],
        axis=1)

    agg1 = _sc_agg(x_aug, src1, dst1, zpad)
    h_aug = _dense(agg1, x_aug, W_l1.T, W_r1.T, b_l1.reshape(1, D), "relu")
    agg2 = _sc_agg(h_aug, src2, dst2, zpad)
    return _dense(agg2, h_aug, W_l2.T, W_r2.T, b_l2.reshape(1, D), "softmax")


# speedup vs baseline: 1.2537x; 1.0198x over previous
import functools

import jax
import jax.numpy as jnp
from jax import lax
from jax.experimental import pallas as pl
from jax.experimental.pallas import tpu as pltpu
from jax.experimental.pallas import tpu_sc as plsc

N = 10000
NP = 10240
E = 320000
D = 128
DA = 144

NC = 2
NS = 16
NW = NC * NS
K = 32
EP = 327680
CHUNKS = EP // K // NW
QCH = 32
NBUF = 4
CH0 = 608
CH1 = 2 * CHUNKS - CH0
ROWS_PER_SUB = NP // NS


def _sc_agg_body(x_hbm, src_hbm, dst_hbm, z_hbm,
                 agg_out,
                 agg_sp, sidx, didx, rows, sem_g, sem_s):
    c = lax.axis_index("c")
    s = lax.axis_index("s")
    wid = c * NS + s
    r0 = s * ROWS_PER_SUB
    npiece = ROWS_PER_SUB // K

    pltpu.sync_copy(z_hbm, rows[0])
    for p in range(npiece):
        pltpu.sync_copy(rows[0], agg_sp.at[pl.ds(r0 + p * K, K)])
    plsc.subcore_barrier()

    def run_ring(first, nslab):
        gd = [None] * NBUF
        sd = [None] * NBUF
        pltpu.sync_copy(src_hbm.at[pl.ds(first, QCH)], sidx[0])
        pltpu.sync_copy(dst_hbm.at[pl.ds(first, QCH)], didx[0])
        for b in range(NBUF):
            gd[b] = pltpu.async_copy(x_hbm.at[sidx[0].at[b]], rows[b], sem_g[b])
        for h in range(nslab):
            p, pn = h % 2, (h + 1) % 2
            if h < nslab - 1:
                base = first + (h + 1) * QCH
                pltpu.sync_copy(src_hbm.at[pl.ds(base, QCH)], sidx[pn])
                pltpu.sync_copy(dst_hbm.at[pl.ds(base, QCH)], didx[pn])
            for i in range(QCH // NBUF):
                for b in range(NBUF):
                    gd[b].wait()
                    sd[b] = pltpu.async_copy(
                        rows[b], agg_sp.at[didx[p].at[NBUF * i + b]],
                        sem_s[b], add=True)
                for b in range(NBUF):
                    sd[b].wait()
                    if i < QCH // NBUF - 1:
                        gd[b] = pltpu.async_copy(
                            x_hbm.at[sidx[p].at[NBUF * (i + 1) + b]], rows[b],
                            sem_g[b])
                    elif h < nslab - 1:
                        gd[b] = pltpu.async_copy(
                            x_hbm.at[sidx[pn].at[b]], rows[b], sem_g[b])

    @pl.when(c == 0)
    def _():
        run_ring(s * CH0, CH0 // QCH)

    @pl.when(c == 1)
    def _():
        run_ring(NS * CH0 + s * CH1, CH1 // QCH)

    plsc.subcore_barrier()
    for p in range(npiece):
        pltpu.sync_copy(agg_sp.at[pl.ds(r0 + p * K, K)], rows[0])
        pltpu.sync_copy(rows[0], agg_out.at[c, pl.ds(r0 + p * K, K)])


_sc_agg = functools.partial(
    pl.kernel,
    out_type=jax.ShapeDtypeStruct((NC, NP, DA), jnp.float32),
    mesh=plsc.VectorSubcoreMesh(core_axis_name="c", subcore_axis_name="s"),
    scratch_types=[
        pltpu.VMEM_SHARED((NP, DA), jnp.float32),
        [pltpu.VMEM((QCH, K), jnp.int32)] * 2,
        [pltpu.VMEM((QCH, K), jnp.int32)] * 2,
        [pltpu.VMEM((K, DA), jnp.float32)] * NBUF,
        [pltpu.SemaphoreType.DMA] * NBUF,
        [pltpu.SemaphoreType.DMA] * NBUF,
    ],
    compiler_params=pltpu.CompilerParams(use_tc_tiling_on_sc=False),
)(_sc_agg_body)


def _dense_body(aggp_ref, x_ref, wl_ref, wr_ref, b_ref, out_ref, *, activation):
    aggp = aggp_ref[0] + aggp_ref[1]
    agg = aggp[:, :D]
    cnt = aggp[:, D:D + 1]
    mean = agg / jnp.maximum(cnt, 1.0)
    y = (jnp.dot(mean, wl_ref[...], preferred_element_type=jnp.float32)
         + jnp.dot(x_ref[:, :D], wr_ref[...], preferred_element_type=jnp.float32)
         + b_ref[...])
    if activation == "relu":
        h = jnp.maximum(y, 0.0)
        col = lax.broadcasted_iota(jnp.int32, (y.shape[0], DA - D), 1)
        out_ref[...] = jnp.concatenate(
            [h, jnp.where(col == 0, 1.0, 0.0)], axis=1)
    else:
        m = jnp.max(y, axis=1, keepdims=True)
        e = jnp.exp(y - m)
        out_ref[...] = e / jnp.sum(e, axis=1, keepdims=True)


def _dense(aggp, x, wl_t, wr_t, b, activation):
    br = 400
    grid = N // br
    d_out = DA if activation == "relu" else D
    return pl.pallas_call(
        functools.partial(_dense_body, activation=activation),
        grid=(grid,),
        in_specs=[
            pl.BlockSpec((NC, br, DA), lambda i: (0, i, 0)),
            pl.BlockSpec((br, DA), lambda i: (i, 0)),
            pl.BlockSpec((D, D), lambda i: (0, 0)),
            pl.BlockSpec((D, D), lambda i: (0, 0)),
            pl.BlockSpec((1, D), lambda i: (0, 0)),
        ],
        out_specs=pl.BlockSpec((br, d_out), lambda i: (i, 0)),
        out_shape=jax.ShapeDtypeStruct((N, d_out), jnp.float32),
    )(aggp, x, wl_t, wr_t, b)


def _pad_edges(edge_index):
    pad = EP - E
    src = jnp.concatenate([edge_index[0], jnp.zeros((pad,), edge_index.dtype)])
    dst = jnp.concatenate([edge_index[1], jnp.full((pad,), N, edge_index.dtype)])
    return src.reshape(NW * CHUNKS, K), dst.reshape(NW * CHUNKS, K)


def kernel(x, edge_index1, edge_index2, W_l1, b_l1, W_r1, W_l2, b_l2, W_r2):
    src1, dst1 = _pad_edges(edge_index1)
    src2, dst2 = _pad_edges(edge_index2)
    zpad = jnp.zeros((K, DA), jnp.float32)
    x_aug = jnp.concatenate(
        [x, jnp.ones((N, 1), jnp.float32), jnp.zeros((N, DA - D - 1), jnp.float32)],
        axis=1)

    agg1 = _sc_agg(x_aug, src1, dst1, zpad)
    h_aug = _dense(agg1, x_aug, W_l1.T, W_r1.T, b_l1.reshape(1, D), "relu")
    agg2 = _sc_agg(h_aug, src2, dst2, zpad)
    return _dense(agg2, h_aug, W_l2.T, W_r2.T, b_l2.reshape(1, D), "softmax")
